# baseline (speedup 1.0000x reference)
import functools

import jax
import jax.numpy as jnp
from jax import lax
from jax.experimental import pallas as pl
from jax.experimental.pallas import tpu as pltpu
from jax.experimental.pallas import tpu_sc as plsc

N = 10000
E = 320000
C = 128
L = 32
HC = C // 2

NC = 2
NS = 16
NW = NC * NS
NP = 10240
EPW = E // NW
RPS = NP // NS

BR = 1000
GRID = N // BR

KD = 400
CHD = EPW // KD

_mesh = plsc.VectorSubcoreMesh(
    core_axis_name="c", subcore_axis_name="s", num_cores=NC, num_subcores=NS
)

_sc_params = pltpu.CompilerParams(use_tc_tiling_on_sc=False)


def _make_deg():
    @functools.partial(
        pl.kernel,
        out_type=jax.ShapeDtypeStruct((NC, NP), jnp.float32),
        mesh=_mesh,
        scratch_types=[
            pltpu.VMEM((CHD, KD), jnp.int32),
            pltpu.VMEM((KD,), jnp.float32),
            pltpu.VMEM_SHARED((NP,), jnp.float32),
        ],
        compiler_params=_sc_params,
    )
    def deg_kernel(dst_hbm, ones_hbm, zeros_hbm, out_hbm, didx, ones_v, acc):
        cid = lax.axis_index("c")
        sid = lax.axis_index("s")
        wid = cid * NS + sid
        pltpu.sync_copy(zeros_hbm, acc.at[pl.ds(sid * RPS, RPS)])
        pltpu.sync_copy(ones_hbm, ones_v)
        pltpu.sync_copy(dst_hbm.at[wid], didx)
        plsc.subcore_barrier()

        def body(i, carry):
            pltpu.sync_copy(ones_v, acc.at[didx.at[i]], add=True)
            return carry

        lax.fori_loop(0, CHD, body, 0)
        plsc.subcore_barrier()
        pltpu.sync_copy(
            acc.at[pl.ds(sid * RPS, RPS)], out_hbm.at[cid, pl.ds(sid * RPS, RPS)]
        )

    return deg_kernel


def _make_prop(HF, nparts, k):
    ch = EPW // k
    assert ch * k == EPW and k % 8 == 0

    @functools.partial(
        pl.kernel,
        out_type=jax.ShapeDtypeStruct((NC, NP, nparts * HF), jnp.float32),
        mesh=_mesh,
        scratch_types=[
            pltpu.VMEM((ch, k), jnp.int32),
            pltpu.VMEM((ch, k), jnp.int32),
            pltpu.VMEM((k, HF), jnp.float32),
            pltpu.VMEM((k, HF), jnp.float32),
            pltpu.VMEM_SHARED((NP, HF), jnp.float32),
            pltpu.SemaphoreType.DMA,
            pltpu.SemaphoreType.DMA,
        ],
        compiler_params=_sc_params,
    )
    def prop_kernel(*refs):
        src_hbm, dst_hbm = refs[0], refs[1]
        ys = refs[2:2 + nparts]
        zeros_hbm = refs[2 + nparts]
        out_hbm = refs[3 + nparts]
        sidx, didx, rows0, rows1, acc, gs0, gs1 = refs[4 + nparts:]
        cid = lax.axis_index("c")
        sid = lax.axis_index("s")
        wid = cid * NS + sid
        pltpu.sync_copy(src_hbm.at[wid], sidx)
        pltpu.sync_copy(dst_hbm.at[wid], didx)

        for p in range(nparts):
            y_hbm = ys[p]
            pltpu.async_copy(y_hbm.at[sidx.at[0]], rows0, gs0)
            pltpu.sync_copy(zeros_hbm, acc.at[pl.ds(sid * RPS, RPS)])
            plsc.subcore_barrier()

            def body(j, carry):
                a = 2 * j
                pltpu.async_copy(y_hbm.at[sidx.at[a + 1]], rows1, gs1)
                pltpu.make_async_copy(y_hbm.at[sidx.at[a]], rows0, gs0).wait()
                pltpu.sync_copy(rows0, acc.at[didx.at[a]], add=True)
                if ch % 2 == 1:
                    pltpu.async_copy(y_hbm.at[sidx.at[a + 2]], rows0, gs0)
                else:
                    @pl.when(a + 2 < ch)
                    def _():
                        pltpu.async_copy(y_hbm.at[sidx.at[a + 2]], rows0, gs0)
                pltpu.make_async_copy(y_hbm.at[sidx.at[a + 1]], rows1, gs1).wait()
                pltpu.sync_copy(rows1, acc.at[didx.at[a + 1]], add=True)
                return carry

            lax.fori_loop(0, ch // 2, body, 0)
            if ch % 2 == 1:
                pltpu.make_async_copy(y_hbm.at[sidx.at[ch - 1]], rows0, gs0).wait()
                pltpu.sync_copy(rows0, acc.at[didx.at[ch - 1]], add=True)

            plsc.subcore_barrier()
            pltpu.sync_copy(
                acc.at[pl.ds(sid * RPS, RPS)],
                out_hbm.at[cid, pl.ds(sid * RPS, RPS), pl.ds(p * HF, HF)],
            )
            plsc.subcore_barrier()

    return prop_kernel


_deg = _make_deg()
_prop64 = _make_prop(HC, 2, 400)
_prop32 = _make_prop(L, 1, 1000)


def _norm_body(deg_ref, x_ref, y0_ref, ylo_ref, yhi_ref, dinv_ref):
    ones21 = jnp.ones((2, 1), jnp.float32)
    deg = lax.dot_general(
        deg_ref[...], ones21, (((0,), (0,)), ((), ())),
        preferred_element_type=jnp.float32,
    ) + 1.0
    dinv = lax.rsqrt(deg)[:N]
    y0 = dinv * x_ref[...]
    y0_ref[...] = y0
    ylo_ref[...] = y0[:, :HC]
    yhi_ref[...] = y0[:, HC:]
    dinv_ref[...] = dinv


def _enc_body(dinv_ref, s_ref, y_ref, wat_ref, ba_ref, wbt_ref, out_ref):
    dinv = dinv_ref[...]
    p = dinv * (s_ref[0] + s_ref[1] + y_ref[...])
    h = jnp.maximum(
        jnp.dot(p, wat_ref[...], preferred_element_type=jnp.float32) + ba_ref[...],
        0.0,
    )
    t = jnp.dot(h, wbt_ref[...], preferred_element_type=jnp.float32)
    out_ref[...] = dinv * t


def _mid_body(dinv_ref, s_ref, y_ref, b_ref, out_ref):
    dinv = dinv_ref[...]
    z = dinv * (s_ref[0] + s_ref[1] + y_ref[...]) + b_ref[...]
    out_ref[...] = dinv * z


def _dec_body(dinv_ref, s_ref, y_ref, wat_ref, ba_ref, wbt_ref,
              y4_ref, ylo_ref, yhi_ref):
    dinv = dinv_ref[...]
    p = dinv * (s_ref[0] + s_ref[1] + y_ref[...])
    h = jnp.maximum(
        jnp.dot(p, wat_ref[...], preferred_element_type=jnp.float32) + ba_ref[...],
        0.0,
    )
    t = dinv * jnp.dot(h, wbt_ref[...], preferred_element_type=jnp.float32)
    y4_ref[...] = t
    ylo_ref[...] = t[:, :HC]
    yhi_ref[...] = t[:, HC:]


def _final_body(dinv_ref, s_ref, y_ref, b_ref, out_ref):
    dinv = dinv_ref[...]
    out_ref[...] = dinv * (s_ref[0] + s_ref[1] + y_ref[...]) + b_ref[...]


def _row_spec(cols):
    return pl.BlockSpec((BR, cols), lambda i: (i, 0))


def _s_spec(cols):
    return pl.BlockSpec((2, BR, cols), lambda i: (0, i, 0))


def _full_spec(*shape):
    return pl.BlockSpec(shape, lambda i: (0,) * len(shape))


_deg_spec = pl.BlockSpec((2, NP), lambda i: (0, 0))


def kernel(x, edge_index, use_neighbors, W1, b1, W2, b2, W3, b3, W4, b4):
    src_p = edge_index[0].reshape(NW, CHD, KD)
    dst_p = edge_index[1].reshape(NW, CHD, KD)
    src_p32 = edge_index[0].reshape(NW, EPW // 1000, 1000)
    dst_p32 = edge_index[1].reshape(NW, EPW // 1000, 1000)

    ones_k = jnp.ones((KD,), jnp.float32)
    zeros1 = jnp.zeros((RPS,), jnp.float32)
    zeros64 = jnp.zeros((RPS, HC), jnp.float32)
    zeros32 = jnp.zeros((RPS, L), jnp.float32)

    f32 = jnp.float32
    o_full = jax.ShapeDtypeStruct((N, C), f32)
    o_32 = jax.ShapeDtypeStruct((N, L), f32)

    deg2 = _deg(dst_p, ones_k, zeros1)

    o_half = jax.ShapeDtypeStruct((N, HC), f32)
    y0, y0lo, y0hi, dinv = pl.pallas_call(
        _norm_body,
        out_shape=[o_full, o_half, o_half, jax.ShapeDtypeStruct((N, 1), f32)],
    )(deg2, x)

    s1 = _prop64(src_p, dst_p, y0lo, y0hi, zeros64)
    y2 = pl.pallas_call(
        _enc_body,
        grid=(GRID,),
        in_specs=[_row_spec(1), _s_spec(C), _row_spec(C),
                  _full_spec(C, 2 * C), _full_spec(1, 2 * C),
                  _full_spec(2 * C, L)],
        out_specs=_row_spec(L),
        out_shape=o_32,
    )(dinv, s1, y0, W1.T, b1[None, :], W2.T)

    s2 = _prop32(src_p32, dst_p32, y2, zeros32)
    y3 = pl.pallas_call(
        _mid_body,
        grid=(GRID,),
        in_specs=[_row_spec(1), _s_spec(L), _row_spec(L), _full_spec(1, L)],
        out_specs=_row_spec(L),
        out_shape=o_32,
    )(dinv, s2, y2, b2[None, :])

    s3 = _prop32(src_p32, dst_p32, y3, zeros32)
    y4, y4lo, y4hi = pl.pallas_call(
        _dec_body,
        grid=(GRID,),
        in_specs=[_row_spec(1), _s_spec(L), _row_spec(L),
                  _full_spec(L, 2 * C), _full_spec(1, 2 * C),
                  _full_spec(2 * C, C)],
        out_specs=[_row_spec(C), _row_spec(HC), _row_spec(HC)],
        out_shape=[o_full, o_half, o_half],
    )(dinv, s3, y3, W3.T, b3[None, :], W4.T)

    s4 = _prop64(src_p, dst_p, y4lo, y4hi, zeros64)
    recon = pl.pallas_call(
        _final_body,
        grid=(GRID,),
        in_specs=[_row_spec(1), _s_spec(C), _row_spec(C), _full_spec(1, C)],
        out_specs=_row_spec(C),
        out_shape=o_full,
    )(dinv, s4, y4, b4[None, :])
    return recon


# speedup vs baseline: 1.0142x; 1.0142x over previous
import functools

import jax
import jax.numpy as jnp
from jax import lax
from jax.experimental import pallas as pl
from jax.experimental.pallas import tpu as pltpu
from jax.experimental.pallas import tpu_sc as plsc

N = 10000
E = 320000
C = 128
L = 32
HC = C // 2

NC = 2
NS = 16
NW = NC * NS
NP = 10240
EPW = E // NW
RPS = NP // NS

BR = 2000
GRID = N // BR

KD = 400
CHD = EPW // KD

_mesh = plsc.VectorSubcoreMesh(
    core_axis_name="c", subcore_axis_name="s", num_cores=NC, num_subcores=NS
)

_sc_params = pltpu.CompilerParams(use_tc_tiling_on_sc=False)


def _make_deg():
    @functools.partial(
        pl.kernel,
        out_type=jax.ShapeDtypeStruct((NC, NP), jnp.float32),
        mesh=_mesh,
        scratch_types=[
            pltpu.VMEM((CHD, KD), jnp.int32),
            pltpu.VMEM((KD,), jnp.float32),
            pltpu.VMEM_SHARED((NP,), jnp.float32),
        ],
        compiler_params=_sc_params,
    )
    def deg_kernel(dst_hbm, ones_hbm, zeros_hbm, out_hbm, didx, ones_v, acc):
        cid = lax.axis_index("c")
        sid = lax.axis_index("s")
        wid = cid * NS + sid
        pltpu.sync_copy(zeros_hbm, acc.at[pl.ds(sid * RPS, RPS)])
        pltpu.sync_copy(ones_hbm, ones_v)
        pltpu.sync_copy(dst_hbm.at[wid], didx)
        plsc.subcore_barrier()

        def body(i, carry):
            pltpu.sync_copy(ones_v, acc.at[didx.at[i]], add=True)
            return carry

        lax.fori_loop(0, CHD, body, 0)
        plsc.subcore_barrier()
        pltpu.sync_copy(
            acc.at[pl.ds(sid * RPS, RPS)], out_hbm.at[cid, pl.ds(sid * RPS, RPS)]
        )

    return deg_kernel


def _make_prop(HF, nparts, k):
    ch = EPW // k
    assert ch * k == EPW and k % 8 == 0

    @functools.partial(
        pl.kernel,
        out_type=jax.ShapeDtypeStruct((NC, NP, nparts * HF), jnp.float32),
        mesh=_mesh,
        scratch_types=[
            pltpu.VMEM((ch, k), jnp.int32),
            pltpu.VMEM((ch, k), jnp.int32),
            pltpu.VMEM((k, HF), jnp.float32),
            pltpu.VMEM((k, HF), jnp.float32),
            pltpu.VMEM_SHARED((NP, HF), jnp.float32),
            pltpu.SemaphoreType.DMA,
            pltpu.SemaphoreType.DMA,
        ],
        compiler_params=_sc_params,
    )
    def prop_kernel(*refs):
        src_hbm, dst_hbm = refs[0], refs[1]
        ys = refs[2:2 + nparts]
        zeros_hbm = refs[2 + nparts]
        out_hbm = refs[3 + nparts]
        sidx, didx, rows0, rows1, acc, gs0, gs1 = refs[4 + nparts:]
        cid = lax.axis_index("c")
        sid = lax.axis_index("s")
        wid = cid * NS + sid
        pltpu.sync_copy(src_hbm.at[wid], sidx)
        pltpu.sync_copy(dst_hbm.at[wid], didx)

        for p in range(nparts):
            y_hbm = ys[p]
            pltpu.async_copy(y_hbm.at[sidx.at[0]], rows0, gs0)
            pltpu.sync_copy(zeros_hbm, acc.at[pl.ds(sid * RPS, RPS)])
            plsc.subcore_barrier()

            def body(j, carry):
                a = 2 * j
                pltpu.async_copy(y_hbm.at[sidx.at[a + 1]], rows1, gs1)
                pltpu.make_async_copy(y_hbm.at[sidx.at[a]], rows0, gs0).wait()
                pltpu.sync_copy(rows0, acc.at[didx.at[a]], add=True)
                if ch % 2 == 1:
                    pltpu.async_copy(y_hbm.at[sidx.at[a + 2]], rows0, gs0)
                else:
                    @pl.when(a + 2 < ch)
                    def _():
                        pltpu.async_copy(y_hbm.at[sidx.at[a + 2]], rows0, gs0)
                pltpu.make_async_copy(y_hbm.at[sidx.at[a + 1]], rows1, gs1).wait()
                pltpu.sync_copy(rows1, acc.at[didx.at[a + 1]], add=True)
                return carry

            lax.fori_loop(0, ch // 2, body, 0)
            if ch % 2 == 1:
                pltpu.make_async_copy(y_hbm.at[sidx.at[ch - 1]], rows0, gs0).wait()
                pltpu.sync_copy(rows0, acc.at[didx.at[ch - 1]], add=True)

            plsc.subcore_barrier()
            pltpu.sync_copy(
                acc.at[pl.ds(sid * RPS, RPS)],
                out_hbm.at[cid, pl.ds(sid * RPS, RPS), pl.ds(p * HF, HF)],
            )
            plsc.subcore_barrier()

    return prop_kernel


_deg = _make_deg()
_prop64 = _make_prop(HC, 2, 400)
_prop32 = _make_prop(L, 1, 1000)


def _norm_body(deg_ref, x_ref, y0_ref, ylo_ref, yhi_ref, dinv_ref):
    ones21 = jnp.ones((2, 1), jnp.float32)
    deg = lax.dot_general(
        deg_ref[...], ones21, (((0,), (0,)), ((), ())),
        preferred_element_type=jnp.float32,
    ) + 1.0
    dinv = lax.rsqrt(deg)[:N]
    y0 = dinv * x_ref[...]
    y0_ref[...] = y0
    ylo_ref[...] = y0[:, :HC]
    yhi_ref[...] = y0[:, HC:]
    dinv_ref[...] = dinv


def _enc_body(dinv_ref, s_ref, y_ref, wat_ref, ba_ref, wbt_ref, out_ref):
    dinv = dinv_ref[...]
    p = dinv * (s_ref[0] + s_ref[1] + y_ref[...])
    h = jnp.maximum(
        jnp.dot(p, wat_ref[...], preferred_element_type=jnp.float32) + ba_ref[...],
        0.0,
    )
    t = jnp.dot(h, wbt_ref[...], preferred_element_type=jnp.float32)
    out_ref[...] = dinv * t


def _mid_body(dinv_ref, s_ref, y_ref, b_ref, out_ref):
    dinv = dinv_ref[...]
    z = dinv * (s_ref[0] + s_ref[1] + y_ref[...]) + b_ref[...]
    out_ref[...] = dinv * z


def _dec_body(dinv_ref, s_ref, y_ref, wat_ref, ba_ref, wbt_ref, y4_ref):
    dinv = dinv_ref[...]
    p = dinv * (s_ref[0] + s_ref[1] + y_ref[...])
    h = jnp.maximum(
        jnp.dot(p, wat_ref[...], preferred_element_type=jnp.float32) + ba_ref[...],
        0.0,
    )
    y4_ref[...] = dinv * jnp.dot(h, wbt_ref[...], preferred_element_type=jnp.float32)


def _final_body(dinv_ref, s_ref, y_ref, b_ref, out_ref):
    dinv = dinv_ref[...]
    out_ref[...] = dinv * (s_ref[0] + s_ref[1] + y_ref[...]) + b_ref[...]


def _row_spec(cols):
    return pl.BlockSpec((BR, cols), lambda i: (i, 0))


def _s_spec(cols):
    return pl.BlockSpec((2, BR, cols), lambda i: (0, i, 0))


def _full_spec(*shape):
    return pl.BlockSpec(shape, lambda i: (0,) * len(shape))


_deg_spec = pl.BlockSpec((2, NP), lambda i: (0, 0))


def kernel(x, edge_index, use_neighbors, W1, b1, W2, b2, W3, b3, W4, b4):
    src_p = edge_index[0].reshape(NW, CHD, KD)
    dst_p = edge_index[1].reshape(NW, CHD, KD)
    src_p32 = edge_index[0].reshape(NW, EPW // 1000, 1000)
    dst_p32 = edge_index[1].reshape(NW, EPW // 1000, 1000)

    ones_k = jnp.ones((KD,), jnp.float32)
    zeros1 = jnp.zeros((RPS,), jnp.float32)
    zeros64 = jnp.zeros((RPS, HC), jnp.float32)
    zeros32 = jnp.zeros((RPS, L), jnp.float32)

    f32 = jnp.float32
    o_full = jax.ShapeDtypeStruct((N, C), f32)
    o_32 = jax.ShapeDtypeStruct((N, L), f32)

    deg2 = _deg(dst_p, ones_k, zeros1)

    o_half = jax.ShapeDtypeStruct((N, HC), f32)
    y0, y0lo, y0hi, dinv = pl.pallas_call(
        _norm_body,
        out_shape=[o_full, o_half, o_half, jax.ShapeDtypeStruct((N, 1), f32)],
    )(deg2, x)

    s1 = _prop64(src_p, dst_p, y0lo, y0hi, zeros64)
    y2 = pl.pallas_call(
        _enc_body,
        grid=(GRID,),
        in_specs=[_row_spec(1), _s_spec(C), _row_spec(C),
                  _full_spec(C, 2 * C), _full_spec(1, 2 * C),
                  _full_spec(2 * C, L)],
        out_specs=_row_spec(L),
        out_shape=o_32,
    )(dinv, s1, y0, W1.T, b1[None, :], W2.T)

    s2 = _prop32(src_p32, dst_p32, y2, zeros32)
    y3 = pl.pallas_call(
        _mid_body,
        grid=(GRID,),
        in_specs=[_row_spec(1), _s_spec(L), _row_spec(L), _full_spec(1, L)],
        out_specs=_row_spec(L),
        out_shape=o_32,
    )(dinv, s2, y2, b2[None, :])

    s3 = _prop32(src_p32, dst_p32, y3, zeros32)
    y4 = pl.pallas_call(
        _dec_body,
        grid=(GRID,),
        in_specs=[_row_spec(1), _s_spec(L), _row_spec(L),
                  _full_spec(L, 2 * C), _full_spec(1, 2 * C),
                  _full_spec(2 * C, C)],
        out_specs=_row_spec(C),
        out_shape=o_full,
    )(dinv, s3, y3, W3.T, b3[None, :], W4.T)

    s4 = _prop64(src_p, dst_p, y4[:, :HC], y4[:, HC:], zeros64)
    recon = pl.pallas_call(
        _final_body,
        grid=(GRID,),
        in_specs=[_row_spec(1), _s_spec(C), _row_spec(C), _full_spec(1, C)],
        out_specs=_row_spec(C),
        out_shape=o_full,
    )(dinv, s4, y4, b4[None, :])
    return recon


# speedup vs baseline: 1.0207x; 1.0064x over previous
import functools

import jax
import jax.numpy as jnp
from jax import lax
from jax.experimental import pallas as pl
from jax.experimental.pallas import tpu as pltpu
from jax.experimental.pallas import tpu_sc as plsc

N = 10000
E = 320000
C = 128
L = 32
HC = C // 2

NC = 2
NS = 16
NW = NC * NS
NP = 10240
EPW = E // NW
RPS = NP // NS

BR = 2000
GRID = N // BR

KD = 2000
CHD = EPW // KD

_mesh = plsc.VectorSubcoreMesh(
    core_axis_name="c", subcore_axis_name="s", num_cores=NC, num_subcores=NS
)

_sc_params = pltpu.CompilerParams(use_tc_tiling_on_sc=False)


def _make_deg():
    @functools.partial(
        pl.kernel,
        out_type=jax.ShapeDtypeStruct((NC, NP), jnp.float32),
        mesh=_mesh,
        scratch_types=[
            pltpu.VMEM((CHD, KD), jnp.int32),
            pltpu.VMEM((KD,), jnp.float32),
            pltpu.VMEM_SHARED((NP,), jnp.float32),
        ],
        compiler_params=_sc_params,
    )
    def deg_kernel(dst_hbm, ones_hbm, zeros_hbm, out_hbm, didx, ones_v, acc):
        cid = lax.axis_index("c")
        sid = lax.axis_index("s")
        wid = cid * NS + sid
        pltpu.sync_copy(zeros_hbm, acc.at[pl.ds(sid * RPS, RPS)])
        pltpu.sync_copy(ones_hbm, ones_v)
        pltpu.sync_copy(dst_hbm.at[wid], didx)
        plsc.subcore_barrier()

        def body(i, carry):
            pltpu.sync_copy(ones_v, acc.at[didx.at[i]], add=True)
            return carry

        lax.fori_loop(0, CHD, body, 0)
        plsc.subcore_barrier()
        pltpu.sync_copy(
            acc.at[pl.ds(sid * RPS, RPS)], out_hbm.at[cid, pl.ds(sid * RPS, RPS)]
        )

    return deg_kernel


def _make_prop(HF, nparts, k):
    ch = EPW // k
    assert ch * k == EPW and k % 8 == 0

    @functools.partial(
        pl.kernel,
        out_type=jax.ShapeDtypeStruct((NC, NP, nparts * HF), jnp.float32),
        mesh=_mesh,
        scratch_types=[
            pltpu.VMEM((ch, k), jnp.int32),
            pltpu.VMEM((ch, k), jnp.int32),
            pltpu.VMEM((k, HF), jnp.float32),
            pltpu.VMEM((k, HF), jnp.float32),
            pltpu.VMEM_SHARED((NP, HF), jnp.float32),
            pltpu.SemaphoreType.DMA,
            pltpu.SemaphoreType.DMA,
        ],
        compiler_params=_sc_params,
    )
    def prop_kernel(*refs):
        src_hbm, dst_hbm = refs[0], refs[1]
        ys = refs[2:2 + nparts]
        zeros_hbm = refs[2 + nparts]
        out_hbm = refs[3 + nparts]
        sidx, didx, rows0, rows1, acc, gs0, gs1 = refs[4 + nparts:]
        cid = lax.axis_index("c")
        sid = lax.axis_index("s")
        wid = cid * NS + sid
        pltpu.sync_copy(src_hbm.at[wid], sidx)
        pltpu.sync_copy(dst_hbm.at[wid], didx)

        for p in range(nparts):
            y_hbm = ys[p]
            pltpu.async_copy(y_hbm.at[sidx.at[0]], rows0, gs0)
            pltpu.sync_copy(zeros_hbm, acc.at[pl.ds(sid * RPS, RPS)])
            plsc.subcore_barrier()

            def body(j, carry):
                a = 2 * j
                pltpu.async_copy(y_hbm.at[sidx.at[a + 1]], rows1, gs1)
                pltpu.make_async_copy(y_hbm.at[sidx.at[a]], rows0, gs0).wait()
                pltpu.sync_copy(rows0, acc.at[didx.at[a]], add=True)
                if ch % 2 == 1:
                    pltpu.async_copy(y_hbm.at[sidx.at[a + 2]], rows0, gs0)
                else:
                    @pl.when(a + 2 < ch)
                    def _():
                        pltpu.async_copy(y_hbm.at[sidx.at[a + 2]], rows0, gs0)
                pltpu.make_async_copy(y_hbm.at[sidx.at[a + 1]], rows1, gs1).wait()
                pltpu.sync_copy(rows1, acc.at[didx.at[a + 1]], add=True)
                return carry

            lax.fori_loop(0, ch // 2, body, 0)
            if ch % 2 == 1:
                pltpu.make_async_copy(y_hbm.at[sidx.at[ch - 1]], rows0, gs0).wait()
                pltpu.sync_copy(rows0, acc.at[didx.at[ch - 1]], add=True)

            plsc.subcore_barrier()
            pltpu.sync_copy(
                acc.at[pl.ds(sid * RPS, RPS)],
                out_hbm.at[cid, pl.ds(sid * RPS, RPS), pl.ds(p * HF, HF)],
            )
            if p < nparts - 1:
                plsc.subcore_barrier()

    return prop_kernel


_deg = _make_deg()
_prop64 = _make_prop(HC, 2, 400)
_prop32 = _make_prop(L, 1, 1000)


def _norm_body(deg_ref, x_ref, y0_ref, ylo_ref, yhi_ref, dinv_ref):
    ones21 = jnp.ones((2, 1), jnp.float32)
    deg = lax.dot_general(
        deg_ref[...], ones21, (((0,), (0,)), ((), ())),
        preferred_element_type=jnp.float32,
    ) + 1.0
    dinv = lax.rsqrt(deg)[:N]
    y0 = dinv * x_ref[...]
    y0_ref[...] = y0
    ylo_ref[...] = y0[:, :HC]
    yhi_ref[...] = y0[:, HC:]
    dinv_ref[...] = dinv


def _enc_body(dinv_ref, s_ref, y_ref, wat_ref, ba_ref, wbt_ref, out_ref):
    dinv = dinv_ref[...]
    p = dinv * (s_ref[0] + s_ref[1] + y_ref[...])
    h = jnp.maximum(
        jnp.dot(p, wat_ref[...], preferred_element_type=jnp.float32) + ba_ref[...],
        0.0,
    )
    t = jnp.dot(h, wbt_ref[...], preferred_element_type=jnp.float32)
    out_ref[...] = dinv * t


def _mid_body(dinv_ref, s_ref, y_ref, b_ref, out_ref):
    dinv = dinv_ref[...]
    z = dinv * (s_ref[0] + s_ref[1] + y_ref[...]) + b_ref[...]
    out_ref[...] = dinv * z


def _dec_body(dinv_ref, s_ref, y_ref, wat_ref, ba_ref, wbt_ref,
              y4_ref, ylo_ref, yhi_ref):
    dinv = dinv_ref[...]
    p = dinv * (s_ref[0] + s_ref[1] + y_ref[...])
    h = jnp.maximum(
        jnp.dot(p, wat_ref[...], preferred_element_type=jnp.float32) + ba_ref[...],
        0.0,
    )
    t = dinv * jnp.dot(h, wbt_ref[...], preferred_element_type=jnp.float32)
    y4_ref[...] = t
    ylo_ref[...] = t[:, :HC]
    yhi_ref[...] = t[:, HC:]


def _final_body(dinv_ref, s_ref, y_ref, b_ref, out_ref):
    dinv = dinv_ref[...]
    out_ref[...] = dinv * (s_ref[0] + s_ref[1] + y_ref[...]) + b_ref[...]


def _row_spec(cols):
    return pl.BlockSpec((BR, cols), lambda i: (i, 0))


def _s_spec(cols):
    return pl.BlockSpec((2, BR, cols), lambda i: (0, i, 0))


def _full_spec(*shape):
    return pl.BlockSpec(shape, lambda i: (0,) * len(shape))


_deg_spec = pl.BlockSpec((2, NP), lambda i: (0, 0))


def kernel(x, edge_index, use_neighbors, W1, b1, W2, b2, W3, b3, W4, b4):
    src_p = edge_index[0].reshape(NW, EPW // 400, 400)
    dst_p = edge_index[1].reshape(NW, EPW // 400, 400)
    dst_pd = edge_index[1].reshape(NW, CHD, KD)
    src_p32 = edge_index[0].reshape(NW, EPW // 1000, 1000)
    dst_p32 = edge_index[1].reshape(NW, EPW // 1000, 1000)

    ones_k = jnp.ones((KD,), jnp.float32)
    zeros1 = jnp.zeros((RPS,), jnp.float32)
    zeros64 = jnp.zeros((RPS, HC), jnp.float32)
    zeros32 = jnp.zeros((RPS, L), jnp.float32)

    f32 = jnp.float32
    o_full = jax.ShapeDtypeStruct((N, C), f32)
    o_32 = jax.ShapeDtypeStruct((N, L), f32)

    deg2 = _deg(dst_pd, ones_k, zeros1)

    o_half = jax.ShapeDtypeStruct((N, HC), f32)
    y0, y0lo, y0hi, dinv = pl.pallas_call(
        _norm_body,
        out_shape=[o_full, o_half, o_half, jax.ShapeDtypeStruct((N, 1), f32)],
    )(deg2, x)

    s1 = _prop64(src_p, dst_p, y0lo, y0hi, zeros64)
    y2 = pl.pallas_call(
        _enc_body,
        grid=(GRID,),
        in_specs=[_row_spec(1), _s_spec(C), _row_spec(C),
                  _full_spec(C, 2 * C), _full_spec(1, 2 * C),
                  _full_spec(2 * C, L)],
        out_specs=_row_spec(L),
        out_shape=o_32,
    )(dinv, s1, y0, W1.T, b1[None, :], W2.T)

    s2 = _prop32(src_p32, dst_p32, y2, zeros32)
    y3 = pl.pallas_call(
        _mid_body,
        grid=(GRID,),
        in_specs=[_row_spec(1), _s_spec(L), _row_spec(L), _full_spec(1, L)],
        out_specs=_row_spec(L),
        out_shape=o_32,
    )(dinv, s2, y2, b2[None, :])

    s3 = _prop32(src_p32, dst_p32, y3, zeros32)
    y4, y4lo, y4hi = pl.pallas_call(
        _dec_body,
        grid=(GRID,),
        in_specs=[_row_spec(1), _s_spec(L), _row_spec(L),
                  _full_spec(L, 2 * C), _full_spec(1, 2 * C),
                  _full_spec(2 * C, C)],
        out_specs=[_row_spec(C), _row_spec(HC), _row_spec(HC)],
        out_shape=[o_full, o_half, o_half],
    )(dinv, s3, y3, W3.T, b3[None, :], W4.T)

    s4 = _prop64(src_p, dst_p, y4lo, y4hi, zeros64)
    recon = pl.pallas_call(
        _final_body,
        grid=(GRID,),
        in_specs=[_row_spec(1), _s_spec(C), _row_spec(C), _full_spec(1, C)],
        out_specs=_row_spec(C),
        out_shape=o_full,
    )(dinv, s4, y4, b4[None, :])
    return recon


# speedup vs baseline: 1.0219x; 1.0012x over previous
import functools

import jax
import jax.numpy as jnp
from jax import lax
from jax.experimental import pallas as pl
from jax.experimental.pallas import tpu as pltpu
from jax.experimental.pallas import tpu_sc as plsc

N = 10000
E = 320000
C = 128
L = 32
HC = C // 2

NC = 2
NS = 16
NW = NC * NS
NP = 10240
EPW = E // NW
RPS = NP // NS

BR = 2000
GRID = N // BR

KD = 2000
CHD = EPW // KD

_mesh = plsc.VectorSubcoreMesh(
    core_axis_name="c", subcore_axis_name="s", num_cores=NC, num_subcores=NS
)

_sc_params = pltpu.CompilerParams(use_tc_tiling_on_sc=False)


def _make_deg():
    @functools.partial(
        pl.kernel,
        out_type=jax.ShapeDtypeStruct((NC, NP), jnp.float32),
        mesh=_mesh,
        scratch_types=[
            pltpu.VMEM((CHD, KD), jnp.int32),
            pltpu.VMEM((KD,), jnp.float32),
            pltpu.VMEM_SHARED((NP,), jnp.float32),
        ],
        compiler_params=_sc_params,
    )
    def deg_kernel(dst_hbm, ones_hbm, zeros_hbm, out_hbm, didx, ones_v, acc):
        cid = lax.axis_index("c")
        sid = lax.axis_index("s")
        wid = cid * NS + sid
        pltpu.sync_copy(zeros_hbm, acc.at[pl.ds(sid * RPS, RPS)])
        pltpu.sync_copy(ones_hbm, ones_v)
        pltpu.sync_copy(dst_hbm.at[wid], didx)
        plsc.subcore_barrier()

        def body(i, carry):
            pltpu.sync_copy(ones_v, acc.at[didx.at[i]], add=True)
            return carry

        lax.fori_loop(0, CHD, body, 0)
        plsc.subcore_barrier()
        pltpu.sync_copy(
            acc.at[pl.ds(sid * RPS, RPS)], out_hbm.at[cid, pl.ds(sid * RPS, RPS)]
        )

    return deg_kernel


def _make_prop(HF, nparts, k):
    ch = EPW // k
    assert ch * k == EPW and k % 8 == 0

    @functools.partial(
        pl.kernel,
        out_type=jax.ShapeDtypeStruct((NC, NP, nparts * HF), jnp.float32),
        mesh=_mesh,
        scratch_types=[
            pltpu.VMEM((ch, k), jnp.int32),
            pltpu.VMEM((ch, k), jnp.int32),
            pltpu.VMEM((k, HF), jnp.float32),
            pltpu.VMEM((k, HF), jnp.float32),
            pltpu.VMEM_SHARED((NP, HF), jnp.float32),
            pltpu.SemaphoreType.DMA,
            pltpu.SemaphoreType.DMA,
        ],
        compiler_params=_sc_params,
    )
    def prop_kernel(*refs):
        src_hbm, dst_hbm = refs[0], refs[1]
        ys = refs[2:2 + nparts]
        zeros_hbm = refs[2 + nparts]
        out_hbm = refs[3 + nparts]
        sidx, didx, rows0, rows1, acc, gs0, gs1 = refs[4 + nparts:]
        cid = lax.axis_index("c")
        sid = lax.axis_index("s")
        wid = cid * NS + sid
        pltpu.sync_copy(src_hbm.at[wid], sidx)
        pltpu.sync_copy(dst_hbm.at[wid], didx)

        for p in range(nparts):
            y_hbm = ys[p]
            pltpu.async_copy(y_hbm.at[sidx.at[0]], rows0, gs0)
            pltpu.sync_copy(zeros_hbm, acc.at[pl.ds(sid * RPS, RPS)])
            plsc.subcore_barrier()

            def body(j, carry):
                a = 2 * j
                pltpu.async_copy(y_hbm.at[sidx.at[a + 1]], rows1, gs1)
                pltpu.make_async_copy(y_hbm.at[sidx.at[a]], rows0, gs0).wait()
                pltpu.sync_copy(rows0, acc.at[didx.at[a]], add=True)
                if ch % 2 == 1:
                    pltpu.async_copy(y_hbm.at[sidx.at[a + 2]], rows0, gs0)
                else:
                    @pl.when(a + 2 < ch)
                    def _():
                        pltpu.async_copy(y_hbm.at[sidx.at[a + 2]], rows0, gs0)
                pltpu.make_async_copy(y_hbm.at[sidx.at[a + 1]], rows1, gs1).wait()
                pltpu.sync_copy(rows1, acc.at[didx.at[a + 1]], add=True)
                return carry

            lax.fori_loop(0, ch // 2, body, 0)
            if ch % 2 == 1:
                pltpu.make_async_copy(y_hbm.at[sidx.at[ch - 1]], rows0, gs0).wait()
                pltpu.sync_copy(rows0, acc.at[didx.at[ch - 1]], add=True)

            plsc.subcore_barrier()
            pltpu.sync_copy(
                acc.at[pl.ds(sid * RPS, RPS)],
                out_hbm.at[cid, pl.ds(sid * RPS, RPS), pl.ds(p * HF, HF)],
            )
            if p < nparts - 1:
                plsc.subcore_barrier()

    return prop_kernel


_deg = _make_deg()
_prop64 = _make_prop(HC, 2, 400)
_prop32 = _make_prop(L, 1, 1000)


def _norm_body(deg_ref, x_ref, y0_ref, ylo_ref, yhi_ref, dinv_ref):
    ones21 = jnp.ones((2, 1), jnp.float32)
    deg = lax.dot_general(
        deg_ref[...], ones21, (((0,), (0,)), ((), ())),
        preferred_element_type=jnp.float32,
    ) + 1.0
    dinv = lax.rsqrt(deg)[:N]
    y0 = dinv * x_ref[...]
    y0_ref[...] = y0
    ylo_ref[...] = y0[:, :HC]
    yhi_ref[...] = y0[:, HC:]
    dinv_ref[...] = dinv


def _enc_body(dinv_ref, s_ref, y_ref, wat_ref, ba_ref, wbt_ref, out_ref):
    dinv = dinv_ref[...]
    p = dinv * (s_ref[0] + s_ref[1] + y_ref[...])
    h = jnp.maximum(
        jnp.dot(p, wat_ref[...], preferred_element_type=jnp.float32) + ba_ref[...],
        0.0,
    )
    t = jnp.dot(h, wbt_ref[...], preferred_element_type=jnp.float32)
    out_ref[...] = dinv * t


def _mid_body(dinv_ref, s_ref, y_ref, b_ref, out_ref):
    dinv = dinv_ref[...]
    z = dinv * (s_ref[0] + s_ref[1] + y_ref[...]) + b_ref[...]
    out_ref[...] = dinv * z


def _dec_body(dinv_ref, s_ref, y_ref, wat_ref, ba_ref, wbt_ref,
              y4_ref, ylo_ref, yhi_ref):
    dinv = dinv_ref[...]
    p = dinv * (s_ref[0] + s_ref[1] + y_ref[...])
    h = jnp.maximum(
        jnp.dot(p, wat_ref[...], preferred_element_type=jnp.float32) + ba_ref[...],
        0.0,
    )
    t = dinv * jnp.dot(h, wbt_ref[...], preferred_element_type=jnp.float32)
    y4_ref[...] = t
    ylo_ref[...] = t[:, :HC]
    yhi_ref[...] = t[:, HC:]


def _final_body(dinv_ref, s_ref, y_ref, b_ref, out_ref):
    dinv = dinv_ref[...]
    out_ref[...] = dinv * (s_ref[0] + s_ref[1] + y_ref[...]) + b_ref[...]


def _row_spec(cols):
    return pl.BlockSpec((BR, cols), lambda i: (i, 0))


def _s_spec(cols):
    return pl.BlockSpec((2, BR, cols), lambda i: (0, i, 0))


def _full_spec(*shape):
    return pl.BlockSpec(shape, lambda i: (0,) * len(shape))


def kernel(x, edge_index, use_neighbors, W1, b1, W2, b2, W3, b3, W4, b4):
    src_p = edge_index[0].reshape(NW, EPW // 400, 400)
    dst_p = edge_index[1].reshape(NW, EPW // 400, 400)
    dst_pd = edge_index[1].reshape(NW, CHD, KD)
    src_p32 = edge_index[0].reshape(NW, EPW // 1000, 1000)
    dst_p32 = edge_index[1].reshape(NW, EPW // 1000, 1000)

    ones_k = jnp.ones((KD,), jnp.float32)
    zeros1 = jnp.zeros((RPS,), jnp.float32)
    zeros64 = jnp.zeros((RPS, HC), jnp.float32)
    zeros32 = jnp.zeros((RPS, L), jnp.float32)

    f32 = jnp.float32
    o_full = jax.ShapeDtypeStruct((N, C), f32)
    o_32 = jax.ShapeDtypeStruct((N, L), f32)

    deg2 = _deg(dst_pd, ones_k, zeros1)

    o_half = jax.ShapeDtypeStruct((N, HC), f32)
    y0, y0lo, y0hi, dinv = pl.pallas_call(
        _norm_body,
        out_shape=[o_full, o_half, o_half, jax.ShapeDtypeStruct((N, 1), f32)],
    )(deg2, x)

    s1 = _prop64(src_p, dst_p, y0lo, y0hi, zeros64)
    y2 = pl.pallas_call(
        _enc_body,
        grid=(GRID,),
        in_specs=[_row_spec(1), _s_spec(C), _row_spec(C),
                  _full_spec(C, 2 * C), _full_spec(1, 2 * C),
                  _full_spec(2 * C, L)],
        out_specs=_row_spec(L),
        out_shape=o_32,
    )(dinv, s1, y0, W1.T, b1[None, :], W2.T)

    s2 = _prop32(src_p32, dst_p32, y2, zeros32)
    y3 = pl.pallas_call(
        _mid_body,
        grid=(GRID,),
        in_specs=[_row_spec(1), _s_spec(L), _row_spec(L), _full_spec(1, L)],
        out_specs=_row_spec(L),
        out_shape=o_32,
    )(dinv, s2, y2, b2[None, :])

    s3 = _prop32(src_p32, dst_p32, y3, zeros32)
    y4, y4lo, y4hi = pl.pallas_call(
        _dec_body,
        grid=(GRID,),
        in_specs=[_row_spec(1), _s_spec(L), _row_spec(L),
                  _full_spec(L, 2 * C), _full_spec(1, 2 * C),
                  _full_spec(2 * C, C)],
        out_specs=[_row_spec(C), _row_spec(HC), _row_spec(HC)],
        out_shape=[o_full, o_half, o_half],
    )(dinv, s3, y3, W3.T, b3[None, :], W4.T)

    s4 = _prop64(src_p, dst_p, y4lo, y4hi, zeros64)
    recon = pl.pallas_call(
        _final_body,
        grid=(GRID,),
        in_specs=[_row_spec(1), _s_spec(C), _row_spec(C), _full_spec(1, C)],
        out_specs=_row_spec(C),
        out_shape=o_full,
    )(dinv, s4, y4, b4[None, :])
    return recon
